# baseline (speedup 1.0000x reference)
import functools

import jax
import jax.numpy as jnp
from jax import lax
from jax.experimental import pallas as pl
from jax.experimental.pallas import tpu as pltpu
from jax.experimental.pallas import tpu_sc as plsc

NC = 2
NS = 16
NW = NC * NS


def _gather_body(table_hbm, idx_hbm, perm_hbm, out_hbm, idx_v, perm_v,
                 *bufs_and_sems, b_per_w, chunk, nbuf):
    rows = bufs_and_sems[:nbuf]
    gsem = bufs_and_sems[nbuf:2 * nbuf]
    osem = bufs_and_sems[2 * nbuf:3 * nbuf]
    n_chunks = b_per_w // chunk

    wid = lax.axis_index("s") * NC + lax.axis_index("c")
    base = wid * b_per_w
    pltpu.sync_copy(idx_hbm.at[pl.ds(base, b_per_w)], idx_v)
    pltpu.sync_copy(perm_hbm.at[pl.ds(base, b_per_w)], perm_v)

    def fire_gather(c):
        b = c % nbuf
        return pltpu.async_copy(
            table_hbm.at[idx_v.at[pl.ds(c * chunk, chunk)]], rows[b], gsem[b])

    gd = [fire_gather(b) for b in range(nbuf)]
    od = [None] * nbuf
    for c in range(n_chunks):
        b = c % nbuf
        gd[b].wait()
        od[b] = pltpu.async_copy(
            rows[b], out_hbm.at[perm_v.at[pl.ds(c * chunk, chunk)]], osem[b])
        nxt = c + nbuf
        if nxt < n_chunks:
            od[b].wait()
            gd[b] = fire_gather(nxt)
    for c in range(max(0, n_chunks - nbuf), n_chunks):
        od[c % nbuf].wait()


def kernel(x, weight):
    B = x.size
    D = weight.shape[1]
    idx = x.reshape(B).astype(jnp.int32)
    pos = lax.iota(jnp.int32, B)
    sidx, perm = lax.sort((idx, pos), num_keys=1)
    b_per_w = B // NW
    chunk = 416
    nbuf = 4

    mesh = plsc.VectorSubcoreMesh(
        core_axis_name="c", subcore_axis_name="s",
        num_cores=NC, num_subcores=NS)

    body = functools.partial(_gather_body, b_per_w=b_per_w, chunk=chunk,
                             nbuf=nbuf)
    out = pl.kernel(
        body,
        out_type=jax.ShapeDtypeStruct((B, D), jnp.float32),
        mesh=mesh,
        scratch_types=(
            [pltpu.VMEM((b_per_w,), jnp.int32),
             pltpu.VMEM((b_per_w,), jnp.int32)]
            + [pltpu.VMEM((chunk, D), jnp.float32) for _ in range(nbuf)]
            + [pltpu.SemaphoreType.DMA for _ in range(2 * nbuf)]
        ),
        compiler_params=pltpu.CompilerParams(use_tc_tiling_on_sc=False),
    )(weight, sidx, perm)
    return out.reshape(x.shape + (D,))


# speedup vs baseline: 1.0028x; 1.0028x over previous
import functools

import jax
import jax.numpy as jnp
from jax import lax
from jax.experimental import pallas as pl
from jax.experimental.pallas import tpu as pltpu
from jax.experimental.pallas import tpu_sc as plsc

NC = 2
NS = 16
NW = NC * NS


def _gather_body(table_hbm, idx_hbm, out_hbm, idx_v, *bufs_and_sems,
                 b_per_w, chunk, nbuf):
    rows = bufs_and_sems[:nbuf]
    gsem = bufs_and_sems[nbuf:2 * nbuf]
    osem = bufs_and_sems[2 * nbuf:3 * nbuf]
    n_chunks = b_per_w // chunk

    wid = lax.axis_index("s") * NC + lax.axis_index("c")
    base = wid * b_per_w
    pltpu.sync_copy(idx_hbm.at[pl.ds(base, b_per_w)], idx_v)

    def fire_gather(c):
        b = c % nbuf
        return pltpu.async_copy(
            table_hbm.at[idx_v.at[pl.ds(c * chunk, chunk)]], rows[b], gsem[b])

    gd = [fire_gather(b) for b in range(nbuf)]
    od = [None] * nbuf
    for c in range(n_chunks):
        b = c % nbuf
        gd[b].wait()
        od[b] = pltpu.async_copy(
            rows[b], out_hbm.at[pl.ds(base + c * chunk, chunk)], osem[b])
        nxt = c + nbuf
        if nxt < n_chunks:
            od[b].wait()
            gd[b] = fire_gather(nxt)
    for c in range(max(0, n_chunks - nbuf), n_chunks):
        od[c % nbuf].wait()


def kernel(x, weight):
    B = x.size
    D = weight.shape[1]
    idx = x.reshape(B).astype(jnp.int32)
    b_per_w = B // NW
    chunk = 416
    nbuf = 4

    mesh = plsc.VectorSubcoreMesh(
        core_axis_name="c", subcore_axis_name="s",
        num_cores=NC, num_subcores=NS)

    body = functools.partial(_gather_body, b_per_w=b_per_w, chunk=chunk,
                             nbuf=nbuf)
    out = pl.kernel(
        body,
        out_type=jax.ShapeDtypeStruct((B, D), jnp.float32),
        mesh=mesh,
        scratch_types=(
            [pltpu.VMEM((b_per_w,), jnp.int32)]
            + [pltpu.VMEM((chunk, D), jnp.float32) for _ in range(nbuf)]
            + [pltpu.SemaphoreType.DMA for _ in range(2 * nbuf)]
        ),
        compiler_params=pltpu.CompilerParams(use_tc_tiling_on_sc=False),
    )(weight, idx)
    return out.reshape(x.shape + (D,))


# speedup vs baseline: 1.6977x; 1.6929x over previous
import jax
import jax.numpy as jnp
from jax import lax
from jax.experimental import pallas as pl
from jax.experimental.pallas import tpu as pltpu
from jax.experimental.pallas import tpu_sc as plsc

NC = 2
NS = 16
NW = NC * NS

BPW = 3328
SEG = 512
FLUSH = 256
NFLUSH = BPW // FLUSH


def _vext(ref, k):
    iota = lax.iota(jnp.int32, 16)
    v = ref[pl.ds((k // 16) * 16, 16)]
    return lax.reduce_max(jnp.where(iota == k % 16, v, 0), (0,))


def _body(wt_hbm, sidx_hbm, perm_hbm, out_hbm,
          sidx_v, perm2d, runs_seg, runs_start, blk0, blk1, rowbuf,
          sA, sB, osem):
    wid = lax.axis_index("s") * NC + lax.axis_index("c")
    base = wid * BPW
    i32 = jnp.int32
    iota = lax.iota(i32, 16)

    pltpu.sync_copy(sidx_hbm.at[pl.ds(base, BPW)], sidx_v)
    for k in range(NFLUSH):
        pltpu.sync_copy(perm_hbm.at[pl.ds(base + k * FLUSH, FLUSH)],
                        perm2d.at[k, 0])

    def p1(v, carry):
        cnt, prev_last = carry
        r = sidx_v[pl.ds(v * 16, 16)]
        seg = lax.shift_right_logical(r, 9)
        pos = jnp.full((16,), v * 16, i32) + iota
        prev_r = plsc.load_gather(sidx_v, [jnp.maximum(pos - 1, 0)])
        prev = lax.shift_right_logical(prev_r, 9)
        first = jnp.logical_or(seg != prev, pos == 0)
        plsc.store_compressed(runs_seg.at[pl.ds(cnt, 16)], seg, mask=first)
        plsc.store_compressed(runs_start.at[pl.ds(cnt, 16)], pos, mask=first)
        n = lax.reduce_max(plsc.all_reduce_population_count(first), (0,))
        last = lax.reduce_max(jnp.where(iota == 15, seg, 0), (0,))
        return cnt + n, last

    nruns, _ = lax.fori_loop(0, BPW // 16, p1,
                             (jnp.int32(0), jnp.int32(-1)))
    runs_start[pl.ds(nruns, 16)] = jnp.full((16,), BPW, i32)

    blks = (blk0, blk1)
    sems = (sA, sB)

    def stage(k, b):
        s = _vext(runs_seg, jnp.minimum(k, nruns - 1))
        pltpu.async_copy(wt_hbm.at[:, pl.ds(s * SEG, SEG)], blks[b], sems[b])

    stage(jnp.int32(0), 0)

    def run_body(k, b):
        pltpu.make_async_copy(wt_hbm.at[:, pl.ds(0, SEG)],
                              blks[b], sems[b]).wait()
        stage(k + 1, 1 - b)
        kc = jnp.minimum(k, nruns - 1)
        s = _vext(runs_seg, kc)
        lo = jnp.where(k < nruns, _vext(runs_start, kc), 0)
        hi = jnp.where(k < nruns, _vext(runs_start, kc + 1), 0)
        sbase = s * SEG

        def hit(j, carry):
            rb = plsc.load_gather(sidx_v, [jnp.full((16,), j, i32)])
            colv = rb - sbase
            jm = j % FLUSH
            jmv = jnp.full((16,), jm, i32)
            for c0 in range(0, 64, 16):
                v = plsc.load_gather(blks[b], [iota + c0, colv])
                plsc.store_scatter(rowbuf, [jmv, iota + c0], v)

            @pl.when(jm == FLUSH - 1)
            def _flush():
                fk = j // FLUSH
                pltpu.async_copy(rowbuf, out_hbm.at[perm2d.at[fk, 0]],
                                 osem).wait()

            return carry

        lax.fori_loop(lo, hi, hit, jnp.int32(0))
        return None

    def pair(kk, carry):
        run_body(2 * kk, 0)
        run_body(2 * kk + 1, 1)
        return carry

    lax.fori_loop(0, (nruns + 1) // 2, pair, jnp.int32(0))
    pltpu.make_async_copy(wt_hbm.at[:, pl.ds(0, SEG)], blks[0], sems[0]).wait()


def kernel(x, weight):
    B = x.size
    D = weight.shape[1]
    idx = x.reshape(B).astype(jnp.int32)
    pos = lax.iota(jnp.int32, B)
    sidx, perm = lax.sort((idx, pos), num_keys=1)
    wt = weight.T

    mesh = plsc.VectorSubcoreMesh(core_axis_name="c", subcore_axis_name="s",
                                  num_cores=NC, num_subcores=NS)
    out = pl.kernel(
        _body,
        out_type=jax.ShapeDtypeStruct((B, 128), jnp.float32),
        mesh=mesh,
        scratch_types=[
            pltpu.VMEM((BPW,), jnp.int32),
            pltpu.VMEM((NFLUSH, 1, FLUSH), jnp.int32),
            pltpu.VMEM((BPW + 32,), jnp.int32),
            pltpu.VMEM((BPW + 32,), jnp.int32),
            pltpu.VMEM((64, SEG), jnp.float32),
            pltpu.VMEM((64, SEG), jnp.float32),
            pltpu.VMEM((FLUSH, 128), jnp.float32),
            pltpu.SemaphoreType.DMA,
            pltpu.SemaphoreType.DMA,
            pltpu.SemaphoreType.DMA,
        ],
        compiler_params=pltpu.CompilerParams(use_tc_tiling_on_sc=True,
                                             needs_layout_passes=False),
    )(wt, sidx, perm)
    return out[:, :D].reshape(x.shape + (D,))


# speedup vs baseline: 2.0436x; 1.2038x over previous
import jax
import jax.numpy as jnp
from jax import lax
from jax.experimental import pallas as pl
from jax.experimental.pallas import tpu as pltpu
from jax.experimental.pallas import tpu_sc as plsc

NC = 2
NS = 16
NW = NC * NS

BPW = 3328
SEG = 512
FLUSH = 256
NFLUSH = BPW // FLUSH


def _vext(ref, k):
    iota = lax.iota(jnp.int32, 16)
    v = ref[pl.ds((k // 16) * 16, 16)]
    return lax.reduce_max(jnp.where(iota == k % 16, v, 0), (0,))


def _body(wt_hbm, sidx_hbm, perm_hbm, out_hbm,
          sidx_v, perm2d, runs_seg, runs_start, blk0, blk1, rowbuf,
          sA, sB, osem):
    wid = lax.axis_index("s") * NC + lax.axis_index("c")
    base = wid * BPW
    i32 = jnp.int32
    iota = lax.iota(i32, 16)

    pltpu.sync_copy(sidx_hbm.at[pl.ds(base, BPW)], sidx_v)
    for k in range(NFLUSH):
        pltpu.sync_copy(perm_hbm.at[pl.ds(base + k * FLUSH, FLUSH)],
                        perm2d.at[k, 0])

    def p1(v, carry):
        cnt, prev_last = carry
        r = sidx_v[pl.ds(v * 16, 16)]
        seg = lax.shift_right_logical(r, 9)
        pos = jnp.full((16,), v * 16, i32) + iota
        prev_r = plsc.load_gather(sidx_v, [jnp.maximum(pos - 1, 0)])
        prev = lax.shift_right_logical(prev_r, 9)
        first = jnp.logical_or(seg != prev, pos == 0)
        plsc.store_compressed(runs_seg.at[pl.ds(cnt, 16)], seg, mask=first)
        plsc.store_compressed(runs_start.at[pl.ds(cnt, 16)], pos, mask=first)
        n = lax.reduce_max(plsc.all_reduce_population_count(first), (0,))
        last = lax.reduce_max(jnp.where(iota == 15, seg, 0), (0,))
        return cnt + n, last

    nruns, _ = lax.fori_loop(0, BPW // 16, p1,
                             (jnp.int32(0), jnp.int32(-1)))
    runs_start[pl.ds(nruns, 16)] = jnp.full((16,), BPW, i32)

    blks = (blk0, blk1)
    sems = (sA, sB)

    def stage(k, b):
        s = _vext(runs_seg, jnp.minimum(k, nruns - 1))
        pltpu.async_copy(wt_hbm.at[:, pl.ds(s * SEG, SEG)], blks[b], sems[b])

    stage(jnp.int32(0), 0)

    def run_body(k, b):
        pltpu.make_async_copy(wt_hbm.at[:, pl.ds(0, SEG)],
                              blks[b], sems[b]).wait()
        stage(k + 1, 1 - b)
        kc = jnp.minimum(k, nruns - 1)
        s = _vext(runs_seg, kc)
        lo = jnp.where(k < nruns, _vext(runs_start, kc), 0)
        hi = jnp.where(k < nruns, _vext(runs_start, kc + 1), 0)
        sbase = s * SEG

        def group(g, carry):
            slo = jnp.maximum(lo, g * FLUSH)
            shi = jnp.minimum(hi, (g + 1) * FLUSH)

            @plsc.parallel_loop(slo, shi, unroll=4)
            def _hits(j):
                rb = plsc.load_gather(sidx_v, [jnp.full((16,), j, i32)])
                colv = rb - sbase
                jmv = jnp.full((16,), j % FLUSH, i32)
                for c0 in range(0, 64, 16):
                    v = plsc.load_gather(blks[b], [iota + c0, colv])
                    plsc.store_scatter(rowbuf, [jmv, iota + c0], v)

            @pl.when(shi == (g + 1) * FLUSH)
            def _flush():
                pltpu.async_copy(rowbuf, out_hbm.at[perm2d.at[g, 0]],
                                 osem).wait()

            return carry

        lax.fori_loop(lo // FLUSH, (hi + FLUSH - 1) // FLUSH, group,
                      jnp.int32(0))
        return None

    def pair(kk, carry):
        run_body(2 * kk, 0)
        run_body(2 * kk + 1, 1)
        return carry

    lax.fori_loop(0, (nruns + 1) // 2, pair, jnp.int32(0))
    pltpu.make_async_copy(wt_hbm.at[:, pl.ds(0, SEG)], blks[0], sems[0]).wait()


def kernel(x, weight):
    B = x.size
    D = weight.shape[1]
    idx = x.reshape(B).astype(jnp.int32)
    pos = lax.iota(jnp.int32, B)
    sidx, perm = lax.sort((idx, pos), num_keys=1)
    wt = weight.T

    mesh = plsc.VectorSubcoreMesh(core_axis_name="c", subcore_axis_name="s",
                                  num_cores=NC, num_subcores=NS)
    out = pl.kernel(
        _body,
        out_type=jax.ShapeDtypeStruct((B, 128), jnp.float32),
        mesh=mesh,
        scratch_types=[
            pltpu.VMEM((BPW,), jnp.int32),
            pltpu.VMEM((NFLUSH, 1, FLUSH), jnp.int32),
            pltpu.VMEM((BPW + 32,), jnp.int32),
            pltpu.VMEM((BPW + 32,), jnp.int32),
            pltpu.VMEM((64, SEG), jnp.float32),
            pltpu.VMEM((64, SEG), jnp.float32),
            pltpu.VMEM((FLUSH, 128), jnp.float32),
            pltpu.SemaphoreType.DMA,
            pltpu.SemaphoreType.DMA,
            pltpu.SemaphoreType.DMA,
        ],
        compiler_params=pltpu.CompilerParams(use_tc_tiling_on_sc=True,
                                             needs_layout_passes=False),
    )(wt, sidx, perm)
    return out[:, :D].reshape(x.shape + (D,))


# speedup vs baseline: 2.0468x; 1.0016x over previous
import jax
import jax.numpy as jnp
from jax import lax
from jax.experimental import pallas as pl
from jax.experimental.pallas import tpu as pltpu
from jax.experimental.pallas import tpu_sc as plsc

NC = 2
NS = 16
NW = NC * NS

BPW = 3328
SEG = 512
FLUSH = 256
NFLUSH = BPW // FLUSH


def _vext(ref, k):
    iota = lax.iota(jnp.int32, 16)
    v = ref[pl.ds((k // 16) * 16, 16)]
    return lax.reduce_max(jnp.where(iota == k % 16, v, 0), (0,))


def _body(wt_hbm, sidx_hbm, perm_hbm, out_hbm,
          sidx_v, perm2d, runs_seg, runs_start, blk0, blk1, rowbuf,
          sA, sB, osem):
    wid = lax.axis_index("s") * NC + lax.axis_index("c")
    base = wid * BPW
    i32 = jnp.int32
    iota = lax.iota(i32, 16)

    pltpu.sync_copy(sidx_hbm.at[pl.ds(base, BPW)], sidx_v)
    for k in range(NFLUSH):
        pltpu.sync_copy(perm_hbm.at[pl.ds(base + k * FLUSH, FLUSH)],
                        perm2d.at[k, 0])

    def p1(v, carry):
        cnt, prev_last = carry
        r = sidx_v[pl.ds(v * 16, 16)]
        seg = lax.shift_right_logical(r, 9)
        pos = jnp.full((16,), v * 16, i32) + iota
        prev_r = plsc.load_gather(sidx_v, [jnp.maximum(pos - 1, 0)])
        prev = lax.shift_right_logical(prev_r, 9)
        first = jnp.logical_or(seg != prev, pos == 0)
        plsc.store_compressed(runs_seg.at[pl.ds(cnt, 16)], seg, mask=first)
        plsc.store_compressed(runs_start.at[pl.ds(cnt, 16)], pos, mask=first)
        n = lax.reduce_max(plsc.all_reduce_population_count(first), (0,))
        last = lax.reduce_max(jnp.where(iota == 15, seg, 0), (0,))
        return cnt + n, last

    nruns, _ = lax.fori_loop(0, BPW // 16, p1,
                             (jnp.int32(0), jnp.int32(-1)))
    runs_start[pl.ds(nruns, 16)] = jnp.full((16,), BPW, i32)

    blks = (blk0, blk1)
    sems = (sA, sB)

    def stage(k, b):
        s = _vext(runs_seg, jnp.minimum(k, nruns - 1))
        pltpu.async_copy(wt_hbm.at[:, pl.ds(s * SEG, SEG)], blks[b], sems[b])

    stage(jnp.int32(0), 0)

    def run_body(k, b):
        pltpu.make_async_copy(wt_hbm.at[:, pl.ds(0, SEG)],
                              blks[b], sems[b]).wait()
        stage(k + 1, 1 - b)
        kc = jnp.minimum(k, nruns - 1)
        s = _vext(runs_seg, kc)
        lo = jnp.where(k < nruns, _vext(runs_start, kc), 0)
        hi = jnp.where(k < nruns, _vext(runs_start, kc + 1), 0)
        sbase = s * SEG

        def group(g, carry):
            slo = jnp.maximum(lo, g * FLUSH)
            shi = jnp.minimum(hi, (g + 1) * FLUSH)

            @plsc.parallel_loop(slo, shi, unroll=8)
            def _hits(j):
                rb = plsc.load_gather(sidx_v, [jnp.full((16,), j, i32)])
                colv = rb - sbase
                jmv = jnp.full((16,), j % FLUSH, i32)
                for c0 in range(0, 64, 16):
                    v = plsc.load_gather(blks[b], [iota + c0, colv])
                    plsc.store_scatter(rowbuf, [jmv, iota + c0], v)

            @pl.when(shi == (g + 1) * FLUSH)
            def _flush():
                pltpu.async_copy(rowbuf, out_hbm.at[perm2d.at[g, 0]],
                                 osem).wait()

            return carry

        lax.fori_loop(lo // FLUSH, (hi + FLUSH - 1) // FLUSH, group,
                      jnp.int32(0))
        return None

    def pair(kk, carry):
        run_body(2 * kk, 0)
        run_body(2 * kk + 1, 1)
        return carry

    lax.fori_loop(0, (nruns + 1) // 2, pair, jnp.int32(0))
    pltpu.make_async_copy(wt_hbm.at[:, pl.ds(0, SEG)], blks[0], sems[0]).wait()


def kernel(x, weight):
    B = x.size
    D = weight.shape[1]
    idx = x.reshape(B).astype(jnp.int32)
    pos = lax.iota(jnp.int32, B)
    sidx, perm = lax.sort((idx, pos), num_keys=1)
    wt = weight.T

    mesh = plsc.VectorSubcoreMesh(core_axis_name="c", subcore_axis_name="s",
                                  num_cores=NC, num_subcores=NS)
    out = pl.kernel(
        _body,
        out_type=jax.ShapeDtypeStruct((B, 128), jnp.float32),
        mesh=mesh,
        scratch_types=[
            pltpu.VMEM((BPW,), jnp.int32),
            pltpu.VMEM((NFLUSH, 1, FLUSH), jnp.int32),
            pltpu.VMEM((BPW + 32,), jnp.int32),
            pltpu.VMEM((BPW + 32,), jnp.int32),
            pltpu.VMEM((64, SEG), jnp.float32),
            pltpu.VMEM((64, SEG), jnp.float32),
            pltpu.VMEM((FLUSH, 128), jnp.float32),
            pltpu.SemaphoreType.DMA,
            pltpu.SemaphoreType.DMA,
            pltpu.SemaphoreType.DMA,
        ],
        compiler_params=pltpu.CompilerParams(use_tc_tiling_on_sc=True,
                                             needs_layout_passes=False),
    )(wt, sidx, perm)
    return out[:, :D].reshape(x.shape + (D,))
